# baseline (speedup 1.0000x reference)
import functools

import jax
import jax.numpy as jnp
from jax import lax
from jax.experimental import pallas as pl
from jax.experimental.pallas import tpu as pltpu
from jax.experimental.pallas import tpu_sc as plsc

B, D = 16384, 128
NC, NS, L = 2, 16, 16
NW = NC * NS
ROWS_PER_W = B // NW
CHUNK = 64
NCHUNKS = ROWS_PER_W // CHUNK
NGRP = CHUNK // L


def _body(mu_h, s2_h, mup_h, s2p_h, lab_h, out_h,
          mu_v, s2_v, mup_v, s2p_v, lab_v, out_v, sem):
    cid = lax.axis_index("c")
    sid = lax.axis_index("s")
    wid = sid * NC + cid
    rbase = wid * ROWS_PER_W

    pltpu.sync_copy(lab_h.at[pl.ds(rbase, ROWS_PER_W)], lab_v)

    ones = jnp.ones((L,), jnp.float32)
    zeros = jnp.zeros((L,), jnp.float32)

    def chunk_body(ci, carry):
        acc0, cnt0 = carry
        e0 = (rbase + ci * CHUNK) * D
        pltpu.sync_copy(mu_h.at[pl.ds(e0, CHUNK * D)], mu_v)
        pltpu.sync_copy(s2_h.at[pl.ds(e0, CHUNK * D)], s2_v)
        pltpu.sync_copy(mup_h.at[pl.ds(e0, CHUNK * D)], mup_v)
        pltpu.sync_copy(s2p_h.at[pl.ds(e0, CHUNK * D)], s2p_v)

        def grp_body(g, carry2):
            acc1, cnt1 = carry2
            labs = lab_v[pl.ds(ci * CHUNK + g * L, L)]
            maskf = jnp.where(labs != 4, ones, zeros)
            cnt1 = cnt1 + maskf

            def row_body(r, acc2):
                idx = jnp.full((L,), r, jnp.int32)
                msk = lax.gather(
                    maskf, idx[:, None],
                    lax.GatherDimensionNumbers(offset_dims=(),
                                               collapsed_slice_dims=(0,),
                                               start_index_map=(0,)),
                    slice_sizes=(1,),
                    mode=lax.GatherScatterMode.PROMISE_IN_BOUNDS)
                off = (g * L + r) * D
                for j in range(D // L):
                    o = off + j * L
                    m = mu_v[pl.ds(o, L)]
                    s2 = s2_v[pl.ds(o, L)]
                    mp = mup_v[pl.ds(o, L)]
                    s2p = s2p_v[pl.ds(o, L)]
                    dm = m - mp
                    term = (1.0 + s2 - s2p
                            - jnp.exp(s2 - s2p)
                            - dm * dm * jnp.exp(-s2p))
                    acc2 = acc2 + msk * term
                return acc2

            acc1 = lax.fori_loop(0, L, row_body, acc1)
            return acc1, cnt1

        return lax.fori_loop(0, NGRP, grp_body, (acc0, cnt0))

    acc, cnt = lax.fori_loop(0, NCHUNKS, chunk_body, (zeros, zeros))
    out_v[pl.ds(0, L)] = acc
    out_v[pl.ds(L, L)] = cnt
    pltpu.sync_copy(out_v, out_h.at[wid])


@jax.jit
def _run(mu, sigma2, mu_pri, sigma2_pri, lab):
    mesh = plsc.VectorSubcoreMesh(core_axis_name="c", subcore_axis_name="s")
    k = pl.kernel(
        _body,
        mesh=mesh,
        out_type=jax.ShapeDtypeStruct((NW, 2 * L), jnp.float32),
        scratch_types=[
            pltpu.VMEM((CHUNK * D,), jnp.float32),
            pltpu.VMEM((CHUNK * D,), jnp.float32),
            pltpu.VMEM((CHUNK * D,), jnp.float32),
            pltpu.VMEM((CHUNK * D,), jnp.float32),
            pltpu.VMEM((ROWS_PER_W,), jnp.int32),
            pltpu.VMEM((2 * L,), jnp.float32),
            pltpu.SemaphoreType.DMA,
        ],
    )
    out = k(mu.reshape(-1), sigma2.reshape(-1), mu_pri.reshape(-1),
            sigma2_pri.reshape(-1), lab)
    total = jnp.sum(out[:, :L])
    n = jnp.sum(out[:, L:])
    loss = -0.5 * total / n
    return jnp.where(n > 0, loss, jnp.float32(0.0))


def kernel(mu, sigma2, mu_pri, sigma2_pri, style_label):
    return _run(mu, sigma2, mu_pri, sigma2_pri,
                style_label.astype(jnp.int32))


# speedup vs baseline: 1.6101x; 1.6101x over previous
import jax
import jax.numpy as jnp
from jax import lax
from jax.experimental import pallas as pl
from jax.experimental.pallas import tpu as pltpu
from jax.experimental.pallas import tpu_sc as plsc

B, D = 16384, 128
NC, NS, L = 2, 16, 16
NW = NC * NS
ROWS_PER_W = B // NW
CHUNK = 64
NCHUNKS = ROWS_PER_W // CHUNK
NGRP = CHUNK // L

_SPLAT_DNUMS = lax.GatherDimensionNumbers(
    offset_dims=(), collapsed_slice_dims=(0,), start_index_map=(0,))


def _body(mu_h, s2_h, mup_h, s2p_h, lab_h, out_h,
          mu_a, s2_a, mup_a, s2p_a, mu_b, s2_b, mup_b, s2p_b,
          lab_v, out_v, sem_a, sem_b):
    cid = lax.axis_index("c")
    sid = lax.axis_index("s")
    wid = sid * NC + cid
    rbase = wid * ROWS_PER_W

    hrefs = (mu_h, s2_h, mup_h, s2p_h)
    bufs_a = (mu_a, s2_a, mup_a, s2p_a)
    bufs_b = (mu_b, s2_b, mup_b, s2p_b)

    pltpu.sync_copy(lab_h.at[pl.ds(rbase, ROWS_PER_W)], lab_v)

    def start(ci, bufs, sem):
        e0 = (rbase + ci * CHUNK) * D
        for h, v in zip(hrefs, bufs):
            pltpu.async_copy(h.at[pl.ds(e0, CHUNK * D)], v, sem)

    def wait(ci, bufs, sem):
        e0 = (rbase + ci * CHUNK) * D
        for h, v in zip(hrefs, bufs):
            pltpu.make_async_copy(h.at[pl.ds(e0, CHUNK * D)], v, sem).wait()

    ones = jnp.ones((L,), jnp.float32)
    zeros = jnp.zeros((L,), jnp.float32)

    def compute(bufs, ci, acc, cnt):
        mu_v, s2_v, mup_v, s2p_v = bufs

        def grp_body(g, carry):
            acc1, cnt1 = carry
            labs = lab_v[pl.ds(ci * CHUNK + g * L, L)]
            maskf = jnp.where(labs != 4, ones, zeros)
            cnt1 = cnt1 + maskf

            def row_body(r, acc2):
                msk = lax.gather(
                    maskf, jnp.full((L, 1), r, jnp.int32), _SPLAT_DNUMS,
                    slice_sizes=(1,),
                    mode=lax.GatherScatterMode.PROMISE_IN_BOUNDS)
                off = (g * L + r) * D
                rowacc = None
                for j in range(D // L):
                    o = off + j * L
                    m = mu_v[pl.ds(o, L)]
                    s2 = s2_v[pl.ds(o, L)]
                    mp = mup_v[pl.ds(o, L)]
                    s2p = s2p_v[pl.ds(o, L)]
                    d = s2 - s2p
                    dm = m - mp
                    term = d - jnp.exp(d) - dm * dm * jnp.exp(-s2p)
                    rowacc = term if rowacc is None else rowacc + term
                return acc2 + msk * rowacc

            acc1 = lax.fori_loop(0, L, row_body, acc1)
            return acc1, cnt1

        return lax.fori_loop(0, NGRP, grp_body, (acc, cnt))

    start(0, bufs_a, sem_a)

    def pair_body(i, carry):
        acc, cnt = carry
        ci0 = 2 * i
        wait(ci0, bufs_a, sem_a)
        start(ci0 + 1, bufs_b, sem_b)
        acc, cnt = compute(bufs_a, ci0, acc, cnt)
        wait(ci0 + 1, bufs_b, sem_b)

        @pl.when(ci0 + 2 < NCHUNKS)
        def _():
            start(ci0 + 2, bufs_a, sem_a)

        acc, cnt = compute(bufs_b, ci0 + 1, acc, cnt)
        return acc, cnt

    acc, cnt = lax.fori_loop(0, NCHUNKS // 2, pair_body, (zeros, zeros))
    out_v[pl.ds(0, L)] = acc
    out_v[pl.ds(L, L)] = cnt
    pltpu.sync_copy(out_v, out_h.at[wid])


@jax.jit
def _run(mu, sigma2, mu_pri, sigma2_pri, lab):
    mesh = plsc.VectorSubcoreMesh(core_axis_name="c", subcore_axis_name="s")
    k = pl.kernel(
        _body,
        mesh=mesh,
        out_type=jax.ShapeDtypeStruct((NW, 2 * L), jnp.float32),
        scratch_types=(
            [pltpu.VMEM((CHUNK * D,), jnp.float32) for _ in range(8)]
            + [pltpu.VMEM((ROWS_PER_W,), jnp.int32),
               pltpu.VMEM((2 * L,), jnp.float32),
               pltpu.SemaphoreType.DMA,
               pltpu.SemaphoreType.DMA]
        ),
    )
    out = k(mu.reshape(-1), sigma2.reshape(-1), mu_pri.reshape(-1),
            sigma2_pri.reshape(-1), lab)
    total = jnp.sum(out[:, :L])
    n = jnp.sum(out[:, L:])
    loss = -0.5 * (total + n * D) / n
    return jnp.where(n > 0, loss, jnp.float32(0.0))


def kernel(mu, sigma2, mu_pri, sigma2_pri, style_label):
    return _run(mu, sigma2, mu_pri, sigma2_pri,
                style_label.astype(jnp.int32))


# speedup vs baseline: 2.2890x; 1.4217x over previous
import jax
import jax.numpy as jnp
from jax.experimental import pallas as pl
from jax.experimental.pallas import tpu as pltpu

B, D = 16384, 128
R = 512
G = B // R


def _tc_body(mu, s2, mup, s2p, lab, acc, cnt):
    i = pl.program_id(0)

    @pl.when(i == 0)
    def _():
        acc[...] = jnp.zeros((8, D), jnp.float32)
        cnt[...] = jnp.zeros((8, D), jnp.float32)

    m = mu[...]
    v = s2[...]
    mp = mup[...]
    vp = s2p[...]
    d = v - vp
    dm = m - mp
    term = d - jnp.exp(d) - dm * dm * jnp.exp(-vp)
    mask = (lab[...] != 4).astype(jnp.float32).reshape(R, 1)
    acc[...] += jnp.sum((term * mask).reshape(R // 8, 8, D), axis=0)
    cnt[...] += jnp.sum(jnp.broadcast_to(mask, (R, D)).reshape(R // 8, 8, D),
                        axis=0)


@jax.jit
def _run(mu, sigma2, mu_pri, sigma2_pri, lab):
    blk = pl.BlockSpec((R, D), lambda i: (i, 0))
    out = pl.pallas_call(
        _tc_body,
        grid=(G,),
        in_specs=[blk, blk, blk, blk, pl.BlockSpec((R,), lambda i: (i,))],
        out_specs=[pl.BlockSpec((8, D), lambda i: (0, 0))] * 2,
        out_shape=[jax.ShapeDtypeStruct((8, D), jnp.float32)] * 2,
    )(mu, sigma2, mu_pri, sigma2_pri, lab)
    total = jnp.sum(out[0])
    n = jnp.sum(out[1]) / D
    loss = -0.5 * (total + n * D) / n
    return jnp.where(n > 0, loss, jnp.float32(0.0))


def kernel(mu, sigma2, mu_pri, sigma2_pri, style_label):
    return _run(mu, sigma2, mu_pri, sigma2_pri,
                style_label.astype(jnp.int32))


# speedup vs baseline: 3.7361x; 1.6322x over previous
import jax
import jax.numpy as jnp
from jax.experimental import pallas as pl
from jax.experimental.pallas import tpu as pltpu

B, D = 16384, 128
R = 2048
G = B // R


def _tc_body(mu, s2, mup, s2p, lab, acc, cnt):
    i = pl.program_id(0)

    @pl.when(i == 0)
    def _():
        acc[...] = jnp.zeros((8, D), jnp.float32)
        cnt[...] = jnp.zeros((8, D), jnp.float32)

    m = mu[...]
    v = s2[...]
    mp = mup[...]
    vp = s2p[...]
    d = v - vp
    dm = m - mp
    term = d - jnp.exp(d) - dm * dm * jnp.exp(-vp)
    mask = (lab[...] != 4).astype(jnp.float32).reshape(R, 1)
    acc[...] += jnp.sum((term * mask).reshape(R // 8, 8, D), axis=0)
    cnt[...] += jnp.sum(jnp.broadcast_to(mask, (R, D)).reshape(R // 8, 8, D),
                        axis=0)


@jax.jit
def _run(mu, sigma2, mu_pri, sigma2_pri, lab):
    blk = pl.BlockSpec((R, D), lambda i: (i, 0))
    out = pl.pallas_call(
        _tc_body,
        grid=(G,),
        in_specs=[blk, blk, blk, blk, pl.BlockSpec((R,), lambda i: (i,))],
        out_specs=[pl.BlockSpec((8, D), lambda i: (0, 0))] * 2,
        out_shape=[jax.ShapeDtypeStruct((8, D), jnp.float32)] * 2,
    )(mu, sigma2, mu_pri, sigma2_pri, lab)
    total = jnp.sum(out[0])
    n = jnp.sum(out[1]) / D
    loss = -0.5 * (total + n * D) / n
    return jnp.where(n > 0, loss, jnp.float32(0.0))


def kernel(mu, sigma2, mu_pri, sigma2_pri, style_label):
    return _run(mu, sigma2, mu_pri, sigma2_pri,
                style_label.astype(jnp.int32))
